# Initial kernel scaffold; baseline (speedup 1.0000x reference)
import jax
import jax.numpy as jnp
from jax.experimental import pallas as pl


def kernel(adv_pc, ori_pc, nearest_indices, weights):
    raise NotImplementedError("write your pallas kernel here")


# speedup vs baseline: 5.6922x; 5.6922x over previous
import functools

import jax
import jax.numpy as jnp
from jax import lax
from jax.experimental import pallas as pl
from jax.experimental.pallas import tpu as pltpu
from jax.experimental.pallas import tpu_sc as plsc

_B, _K, _KNN = 16, 8192, 16
_L = 16
_NC, _NS = 2, 16
_NW = _NC * _NS
_IDX_PER_W = _B * _K * _KNN // _NW
_CK = 3 * _K

_mesh = plsc.VectorSubcoreMesh(core_axis_name="c", subcore_axis_name="s")


@functools.partial(
    pl.kernel,
    out_type=jax.ShapeDtypeStruct((_NW, _L), jnp.float32),
    mesh=_mesh,
    scratch_types=[
        pltpu.VMEM((_CK,), jnp.float32),
        pltpu.VMEM((_CK,), jnp.float32),
        pltpu.VMEM((_K,), jnp.float32),
        pltpu.VMEM((_IDX_PER_W,), jnp.int32),
        pltpu.VMEM((_L,), jnp.float32),
        pltpu.SemaphoreType.DMA,
    ],
    compiler_params=pltpu.CompilerParams(needs_layout_passes=False),
)
def _sc_gather_sum(adv_hbm, ori_hbm, idx_hbm, out_hbm,
                   adv_v, ori_v, s_v, idx_v, acc_v, sem):
    wid = lax.axis_index("s") * _NC + lax.axis_index("c")
    b = wid // 2

    idx_cp = pltpu.async_copy(idx_hbm.at[wid], idx_v, sem)
    pltpu.sync_copy(adv_hbm.at[b], adv_v)
    pltpu.sync_copy(ori_hbm.at[b], ori_v)

    def sbody(j, carry):
        off = j * _L
        d0 = adv_v[pl.ds(off, _L)] - ori_v[pl.ds(off, _L)]
        d1 = adv_v[pl.ds(_K + off, _L)] - ori_v[pl.ds(_K + off, _L)]
        d2 = adv_v[pl.ds(2 * _K + off, _L)] - ori_v[pl.ds(2 * _K + off, _L)]
        s_v[pl.ds(off, _L)] = d0 * d0 + d1 * d1 + d2 * d2
        return carry

    lax.fori_loop(0, _K // _L, sbody, 0, unroll=4)

    idx_cp.wait()

    def gbody(t, acc):
        iv = idx_v[pl.ds(t * _L, _L)]
        return acc + plsc.load_gather(s_v, [iv])

    acc = lax.fori_loop(0, _IDX_PER_W // _L, gbody,
                        jnp.zeros((_L,), jnp.float32), unroll=8)
    acc_v[...] = acc
    pltpu.sync_copy(acc_v, out_hbm.at[wid])


def _tc_reduce_body(p_ref, w_ref, o_ref):
    o_ref[...] = jnp.sum(p_ref[...] * w_ref[...], keepdims=True)


_tc_reduce = pl.pallas_call(
    _tc_reduce_body,
    out_shape=jax.ShapeDtypeStruct((1, 1), jnp.float32),
)


def kernel(adv_pc, ori_pc, nearest_indices, weights):
    adv2 = adv_pc.reshape(_B, _CK)
    ori2 = ori_pc.reshape(_B, _CK)
    idx2 = nearest_indices.reshape(_NW, _IDX_PER_W)
    partials = _sc_gather_sum(adv2, ori2, idx2)
    wrow = jnp.repeat(weights * (1.0 / _B), 2)[:, None]
    out = _tc_reduce(partials, wrow)
    return out[0, 0]


# speedup vs baseline: 5.7939x; 1.0179x over previous
import functools

import jax
import jax.numpy as jnp
from jax import lax
from jax.experimental import pallas as pl
from jax.experimental.pallas import tpu as pltpu
from jax.experimental.pallas import tpu_sc as plsc

_B, _K, _KNN = 16, 8192, 16
_L = 16
_NC, _NS = 2, 16
_NW = _NC * _NS
_IDX_PER_W = _B * _K * _KNN // _NW
_CK = 3 * _K

_mesh = plsc.VectorSubcoreMesh(core_axis_name="c", subcore_axis_name="s")


@functools.partial(
    pl.kernel,
    out_type=jax.ShapeDtypeStruct((_NW, _L), jnp.float32),
    mesh=_mesh,
    scratch_types=[
        pltpu.VMEM((_CK,), jnp.float32),
        pltpu.VMEM((_CK,), jnp.float32),
        pltpu.VMEM((_K,), jnp.float32),
        pltpu.VMEM((_IDX_PER_W,), jnp.int32),
        pltpu.VMEM((_L,), jnp.float32),
        pltpu.SemaphoreType.DMA,
    ],
    compiler_params=pltpu.CompilerParams(needs_layout_passes=False),
)
def _sc_gather_sum(adv_hbm, ori_hbm, idx_hbm, out_hbm,
                   adv_v, ori_v, s_v, idx_v, acc_v, sem):
    wid = lax.axis_index("s") * _NC + lax.axis_index("c")
    b = wid // 2
    h = wid % 2

    idx_cp = pltpu.async_copy(idx_hbm.at[b, pl.ds(h * _IDX_PER_W, _IDX_PER_W)],
                              idx_v, sem)
    pltpu.sync_copy(adv_hbm.at[b], adv_v)
    pltpu.sync_copy(ori_hbm.at[b], ori_v)

    def sbody(j, carry):
        off = j * _L
        d0 = adv_v[pl.ds(off, _L)] - ori_v[pl.ds(off, _L)]
        d1 = adv_v[pl.ds(_K + off, _L)] - ori_v[pl.ds(_K + off, _L)]
        d2 = adv_v[pl.ds(2 * _K + off, _L)] - ori_v[pl.ds(2 * _K + off, _L)]
        s_v[pl.ds(off, _L)] = d0 * d0 + d1 * d1 + d2 * d2
        return carry

    lax.fori_loop(0, _K // _L, sbody, 0, unroll=4)

    idx_cp.wait()

    def gbody(t, acc):
        iv = idx_v[pl.ds(t * _L, _L)]
        return acc + plsc.load_gather(s_v, [iv])

    acc = lax.fori_loop(0, _IDX_PER_W // _L, gbody,
                        jnp.zeros((_L,), jnp.float32), unroll=8)
    acc_v[...] = acc
    pltpu.sync_copy(acc_v, out_hbm.at[wid])


def _tc_reduce_body(p_ref, w_ref, o_ref):
    o_ref[...] = jnp.sum(p_ref[...] * w_ref[...], keepdims=True)


_tc_reduce = pl.pallas_call(
    _tc_reduce_body,
    out_shape=jax.ShapeDtypeStruct((1, 1), jnp.float32),
)


def kernel(adv_pc, ori_pc, nearest_indices, weights):
    adv2 = adv_pc.reshape(_B, _CK)
    ori2 = ori_pc.reshape(_B, _CK)
    idx2 = nearest_indices.reshape(_B, _K * _KNN)
    partials = _sc_gather_sum(adv2, ori2, idx2)
    wrow = jnp.repeat(weights * (1.0 / _B), 2)[:, None]
    out = _tc_reduce(partials, wrow)
    return out[0, 0]


# speedup vs baseline: 8.6232x; 1.4883x over previous
import functools

import jax
import jax.numpy as jnp
from jax import lax
from jax.experimental import pallas as pl
from jax.experimental.pallas import tpu as pltpu
from jax.experimental.pallas import tpu_sc as plsc

_B, _K, _KNN = 16, 8192, 16
_L = 16
_NC, _NS = 2, 16
_NW = _NC * _NS
_IDX_PER_W = _B * _K * _KNN // _NW
_ROWS_PER_W = _IDX_PER_W // _KNN

_mesh = plsc.VectorSubcoreMesh(core_axis_name="c", subcore_axis_name="s")


@functools.partial(
    pl.kernel,
    out_type=jax.ShapeDtypeStruct((_NW, _L), jnp.float32),
    mesh=_mesh,
    scratch_types=[
        pltpu.VMEM((3, _K), jnp.float32),
        pltpu.VMEM((3, _K), jnp.float32),
        pltpu.VMEM((_K,), jnp.float32),
        pltpu.VMEM((_ROWS_PER_W, _KNN), jnp.int32),
        pltpu.VMEM((_L,), jnp.float32),
        pltpu.SemaphoreType.DMA,
    ],
    compiler_params=pltpu.CompilerParams(
        needs_layout_passes=False, use_tc_tiling_on_sc=False),
)
def _sc_gather_sum(adv_hbm, ori_hbm, idx_hbm, out_hbm,
                   adv_v, ori_v, s_v, idx_v, acc_v, sem):
    wid = lax.axis_index("s") * _NC + lax.axis_index("c")
    b = wid // 2
    h = wid % 2

    idx_cp = pltpu.async_copy(
        idx_hbm.at[b, pl.ds(h * _ROWS_PER_W, _ROWS_PER_W)], idx_v, sem)
    pltpu.sync_copy(adv_hbm.at[b], adv_v)
    pltpu.sync_copy(ori_hbm.at[b], ori_v)

    def sbody(j, carry):
        off = j * _L
        d0 = adv_v[0, pl.ds(off, _L)] - ori_v[0, pl.ds(off, _L)]
        d1 = adv_v[1, pl.ds(off, _L)] - ori_v[1, pl.ds(off, _L)]
        d2 = adv_v[2, pl.ds(off, _L)] - ori_v[2, pl.ds(off, _L)]
        s_v[pl.ds(off, _L)] = d0 * d0 + d1 * d1 + d2 * d2
        return carry

    lax.fori_loop(0, _K // _L, sbody, 0, unroll=4)

    idx_cp.wait()

    def gbody(t, acc):
        iv = idx_v[t, :]
        return acc + plsc.load_gather(s_v, [iv])

    acc = lax.fori_loop(0, _ROWS_PER_W, gbody,
                        jnp.zeros((_L,), jnp.float32), unroll=8)
    acc_v[...] = acc
    pltpu.sync_copy(acc_v, out_hbm.at[wid])


def _tc_reduce_body(p_ref, w_ref, o_ref):
    o_ref[...] = jnp.sum(p_ref[...] * w_ref[...], keepdims=True)


_tc_reduce = pl.pallas_call(
    _tc_reduce_body,
    out_shape=jax.ShapeDtypeStruct((1, 1), jnp.float32),
)


def kernel(adv_pc, ori_pc, nearest_indices, weights):
    partials = _sc_gather_sum(adv_pc, ori_pc, nearest_indices)
    wrow = jnp.repeat(weights * (1.0 / _B), 2)[:, None]
    out = _tc_reduce(partials, wrow)
    return out[0, 0]


# speedup vs baseline: 16.4491x; 1.9075x over previous
import functools

import jax
import jax.numpy as jnp
from jax import lax
from jax.experimental import pallas as pl
from jax.experimental.pallas import tpu as pltpu
from jax.experimental.pallas import tpu_sc as plsc

_B, _K, _KNN = 16, 8192, 16
_L = 16
_NC, _NS = 2, 16
_NW = _NC * _NS
_IDX_PER_W = _B * _K * _KNN // _NW
_ROWS_PER_W = _IDX_PER_W // _KNN

_mesh = plsc.VectorSubcoreMesh(core_axis_name="c", subcore_axis_name="s")


@functools.partial(
    pl.kernel,
    out_type=jax.ShapeDtypeStruct((_NW, _L), jnp.float32),
    mesh=_mesh,
    scratch_types=[
        pltpu.VMEM((3, _K), jnp.float32),
        pltpu.VMEM((3, _K), jnp.float32),
        pltpu.VMEM((_K,), jnp.float32),
        pltpu.VMEM((_KNN // 2, _K), jnp.int32),
        pltpu.VMEM((_L,), jnp.float32),
        pltpu.SemaphoreType.DMA,
    ],
    compiler_params=pltpu.CompilerParams(
        needs_layout_passes=False, use_tc_tiling_on_sc=False),
)
def _sc_gather_sum(adv_hbm, ori_hbm, idx_hbm, out_hbm,
                   adv_v, ori_v, s_v, idx_v, acc_v, sem):
    wid = lax.axis_index("s") * _NC + lax.axis_index("c")
    b = wid // 2
    h = wid % 2

    idx_cp = pltpu.async_copy(
        idx_hbm.at[b, pl.ds(h * (_KNN // 2), _KNN // 2)], idx_v, sem)
    pltpu.sync_copy(adv_hbm.at[b], adv_v)
    pltpu.sync_copy(ori_hbm.at[b], ori_v)

    def sbody(j, carry):
        off = j * _L
        d0 = adv_v[0, pl.ds(off, _L)] - ori_v[0, pl.ds(off, _L)]
        d1 = adv_v[1, pl.ds(off, _L)] - ori_v[1, pl.ds(off, _L)]
        d2 = adv_v[2, pl.ds(off, _L)] - ori_v[2, pl.ds(off, _L)]
        s_v[pl.ds(off, _L)] = d0 * d0 + d1 * d1 + d2 * d2
        return carry

    lax.fori_loop(0, _K // _L, sbody, 0, unroll=4)

    idx_cp.wait()

    def gbody(j, acc):
        off = j * _L
        for n in range(_KNN // 2):
            iv = idx_v[n, pl.ds(off, _L)]
            acc = acc + plsc.load_gather(s_v, [iv])
        return acc

    acc = lax.fori_loop(0, _K // _L, gbody,
                        jnp.zeros((_L,), jnp.float32), unroll=2)
    acc_v[...] = acc
    pltpu.sync_copy(acc_v, out_hbm.at[wid])


def _tc_reduce_body(p_ref, w_ref, o_ref):
    o_ref[...] = jnp.sum(p_ref[...] * w_ref[...], keepdims=True)


_tc_reduce = pl.pallas_call(
    _tc_reduce_body,
    out_shape=jax.ShapeDtypeStruct((1, 1), jnp.float32),
)


def kernel(adv_pc, ori_pc, nearest_indices, weights):
    idx_t = jnp.swapaxes(nearest_indices, 1, 2)
    partials = _sc_gather_sum(adv_pc, ori_pc, idx_t)
    wrow = jnp.repeat(weights * (1.0 / _B), 2)[:, None]
    out = _tc_reduce(partials, wrow)
    return out[0, 0]
